# Initial kernel scaffold; baseline (speedup 1.0000x reference)
import jax
import jax.numpy as jnp
from jax.experimental import pallas as pl


def kernel(atom_pos, x, edge_index, edge_attr, emb_table, mat_W, mat_b, emb_W, emb_b, sbf_Ws, src_Ws, src_bs, eattr_Ws, eattr_bs, upd_Ws, upd_bs, rbf_out_W, out_W, out_b, a1_W, a1_b, a2_W, a2_b):
    raise NotImplementedError("write your pallas kernel here")


# speedup vs baseline: 6.5638x; 6.5638x over previous
import functools

import jax
import jax.numpy as jnp
from jax import lax
from jax.experimental import pallas as pl
from jax.experimental.pallas import tpu as pltpu
from jax.experimental.pallas import tpu_sc as plsc

CUTOFF = 5.0
P_EXP = 5
SBF_DIM = 7
RBF_DIM = 16


def _silu(v):
    return v * (1.0 / (1.0 + jnp.exp(-v)))


def _sc_gather(table, idx):
    V, D = table.shape
    B = idx.shape[0]
    info = plsc.get_sparse_core_info()
    NC, NS = info.num_cores, info.num_subcores
    NW = NC * NS
    CH = 128
    assert B % (NW * CH) == 0 and D % 16 == 0
    b_per_w = B // NW
    n_ch = b_per_w // CH
    mesh = plsc.VectorSubcoreMesh(core_axis_name="c", subcore_axis_name="s")

    @functools.partial(
        pl.kernel,
        mesh=mesh,
        out_type=jax.ShapeDtypeStruct((B, D), table.dtype),
        compiler_params=pltpu.CompilerParams(use_tc_tiling_on_sc=False),
        scratch_types=[
            pltpu.VMEM((CH,), jnp.int32),
            pltpu.VMEM((CH, D), table.dtype),
            pltpu.SemaphoreType.DMA,
        ],
    )
    def k(table_hbm, idx_hbm, out_hbm, idx_v, rows_v, sem):
        wid = lax.axis_index("s") * NC + lax.axis_index("c")
        base = wid * b_per_w

        def body(c, carry):
            off = base + c * CH
            pltpu.sync_copy(idx_hbm.at[pl.ds(off, CH)], idx_v)
            pltpu.async_copy(table_hbm.at[idx_v], rows_v, sem).wait()
            pltpu.sync_copy(rows_v, out_hbm.at[pl.ds(off, CH)])
            return carry

        lax.fori_loop(0, n_ch, body, 0)

    return k(table, idx)


def _stage1(ea_c, pos_s, pos_d, meta_c, mat_Wp, mat_b8, emb_W, emb_b8):
    E = ea_c.shape[0]
    BLK = 512

    def body(ea_ref, ps_ref, pd_ref, mt_ref, W1_ref, b1_ref, W2_ref, b2_ref,
             h0_ref, geo_ref, rb_ref):
        ps = ps_ref[...]
        pd = pd_ref[...]
        r = pd - ps
        d2 = jnp.sum(r * r, axis=1, keepdims=True)
        d = jnp.sqrt(d2)
        inv = lax.rsqrt(jnp.maximum(d2, 1e-18))
        u = r * inv
        geo_ref[...] = u + mt_ref[...]

        xx = d * (1.0 / CUTOFF)
        p = float(P_EXP)
        c1 = -(p + 1.0) * (p + 2.0) / 2.0
        c2 = p * (p + 2.0)
        c3 = -p * (p + 1.0) / 2.0
        x2 = xx * xx
        x4 = x2 * x2
        x5 = x4 * xx
        env = 1.0 + c1 * x5 + c2 * x5 * xx + c3 * x5 * x2
        env = jnp.where(xx < 1.0, env, 0.0)

        n = lax.broadcasted_iota(
            jnp.int32, (BLK, RBF_DIM), 1).astype(jnp.float32) + 1.0
        rb = (jnp.sqrt(2.0 / CUTOFF)
              * jnp.sin(n * (jnp.pi / CUTOFF) * d)
              / jnp.maximum(d, 1e-9)) * env
        rb_ref[...] = rb

        neo = ea_ref[...] * env
        t1 = _silu(jnp.dot(neo, W1_ref[...],
                           preferred_element_type=jnp.float32)
                   + b1_ref[0:1, :])
        h0_ref[...] = _silu(jnp.dot(t1, W2_ref[...],
                                    preferred_element_type=jnp.float32)
                            + b2_ref[0:1, :])

    KP = mat_Wp.shape[0]
    return pl.pallas_call(
        body,
        grid=(E // BLK,),
        in_specs=[
            pl.BlockSpec((BLK, KP), lambda i: (i, 0)),
            pl.BlockSpec((BLK, 16), lambda i: (i, 0)),
            pl.BlockSpec((BLK, 16), lambda i: (i, 0)),
            pl.BlockSpec((BLK, 16), lambda i: (i, 0)),
            pl.BlockSpec((KP, 256), lambda i: (0, 0)),
            pl.BlockSpec((8, 256), lambda i: (0, 0)),
            pl.BlockSpec((256, 256), lambda i: (0, 0)),
            pl.BlockSpec((8, 256), lambda i: (0, 0)),
        ],
        out_specs=[
            pl.BlockSpec((BLK, 256), lambda i: (i, 0)),
            pl.BlockSpec((BLK, 16), lambda i: (i, 0)),
            pl.BlockSpec((BLK, 16), lambda i: (i, 0)),
        ],
        out_shape=[
            jax.ShapeDtypeStruct((E, 256), jnp.float32),
            jax.ShapeDtypeStruct((E, 16), jnp.float32),
            jax.ShapeDtypeStruct((E, 16), jnp.float32),
        ],
        compiler_params=pltpu.CompilerParams(
            dimension_semantics=("arbitrary",)),
    )(ea_c, pos_s, pos_d, meta_c, mat_Wp, mat_b8, emb_W, emb_b8)


def _ab(h, emb_c, src_W, src_b8, eattr_W, eattr_b8):
    E = h.shape[0]
    BLK = 512

    def body(h_ref, e_ref, Ws_ref, bs_ref, We_ref, be_ref, A_ref, B_ref):
        A_ref[...] = _silu(jnp.dot(h_ref[...], Ws_ref[...],
                                   preferred_element_type=jnp.float32)
                           + bs_ref[0:1, :])
        B_ref[...] = _silu(jnp.dot(e_ref[...], We_ref[...],
                                   preferred_element_type=jnp.float32)
                           + be_ref[0:1, :])

    return pl.pallas_call(
        body,
        grid=(E // BLK,),
        in_specs=[
            pl.BlockSpec((BLK, 256), lambda i: (i, 0)),
            pl.BlockSpec((BLK, 128), lambda i: (i, 0)),
            pl.BlockSpec((256, 128), lambda i: (0, 0)),
            pl.BlockSpec((8, 128), lambda i: (0, 0)),
            pl.BlockSpec((128, 128), lambda i: (0, 0)),
            pl.BlockSpec((8, 128), lambda i: (0, 0)),
        ],
        out_specs=[
            pl.BlockSpec((BLK, 128), lambda i: (i, 0)),
            pl.BlockSpec((BLK, 128), lambda i: (i, 0)),
        ],
        out_shape=[
            jax.ShapeDtypeStruct((E, 128), jnp.float32),
            jax.ShapeDtypeStruct((E, 128), jnp.float32),
        ],
        compiler_params=pltpu.CompilerParams(
            dimension_semantics=("arbitrary",)),
    )(h, emb_c, src_W, src_b8, eattr_W, eattr_b8)


def _triplet(chunk_start, n_chunks, geo_c, Bm, A_in, rb_in, geoT_in, sbf_W2):
    E = Bm.shape[0]
    OB = 128

    def body(cs_ref, nc_ref, geo_ref, B_ref, A_hbm, rb_hbm, gT_hbm, W2_ref,
             out_ref, A_v, rb_v, gT_v, acc, s1, s2, s3):
        i = pl.program_id(0)
        c0 = cs_ref[i]
        nc = nc_ref[i]
        acc[...] = jnp.zeros_like(acc)
        geo = geo_ref[...]
        uo0 = geo[:, 0:1]
        uo1 = geo[:, 1:2]
        uo2 = geo[:, 2:3]
        onode = geo[:, 3:4]
        oatom = geo[:, 4:5]

        def chunk(c, carry):
            base = (c0 + c) * 128
            cpA = pltpu.make_async_copy(A_hbm.at[pl.ds(base, 128), :], A_v, s1)
            cpR = pltpu.make_async_copy(rb_hbm.at[pl.ds(base, 128), :], rb_v, s2)
            cpG = pltpu.make_async_copy(gT_hbm.at[:, pl.ds(base, 128)], gT_v, s3)
            cpA.start()
            cpR.start()
            cpG.start()
            cpA.wait()
            cpR.wait()
            cpG.wait()
            gT = gT_v[...]
            ui0 = gT[0:1, :]
            ui1 = gT[1:2, :]
            ui2 = gT[2:3, :]
            iatom = gT[3:4, :]
            innode = gT[4:5, :]
            u = -(uo0 * ui0 + uo1 * ui1 + uo2 * ui2)
            u = jnp.clip(u, -1.0, 1.0)
            mask = jnp.where((onode == innode) & (iatom != oatom), 1.0, 0.0)
            C = jnp.dot(rb_v[...], W2_ref[...],
                        preferred_element_type=jnp.float32)
            A = A_v[...]
            t_prev = mask
            t_cur = u * mask
            a = acc[...]
            a = a + jnp.dot(t_prev, A * C[:, 0:128],
                            preferred_element_type=jnp.float32)
            for l in range(1, SBF_DIM):
                a = a + jnp.dot(t_cur, A * C[:, l * 128:(l + 1) * 128],
                                preferred_element_type=jnp.float32)
                t_next = 2.0 * u * t_cur - t_prev
                t_prev = t_cur
                t_cur = t_next
            acc[...] = a
            return carry

        lax.fori_loop(0, nc, chunk, 0)
        out_ref[...] = B_ref[...] * acc[...]

    grid_spec = pltpu.PrefetchScalarGridSpec(
        num_scalar_prefetch=2,
        grid=(E // OB,),
        in_specs=[
            pl.BlockSpec((OB, 16), lambda i, *_: (i, 0)),
            pl.BlockSpec((OB, 128), lambda i, *_: (i, 0)),
            pl.BlockSpec(memory_space=pl.ANY),
            pl.BlockSpec(memory_space=pl.ANY),
            pl.BlockSpec(memory_space=pl.ANY),
            pl.BlockSpec((16, SBF_DIM * 128), lambda i, *_: (0, 0)),
        ],
        out_specs=pl.BlockSpec((OB, 128), lambda i, *_: (i, 0)),
        scratch_shapes=[
            pltpu.VMEM((128, 128), jnp.float32),
            pltpu.VMEM((128, 16), jnp.float32),
            pltpu.VMEM((16, 128), jnp.float32),
            pltpu.VMEM((OB, 128), jnp.float32),
            pltpu.SemaphoreType.DMA,
            pltpu.SemaphoreType.DMA,
            pltpu.SemaphoreType.DMA,
        ],
    )
    return pl.pallas_call(
        body,
        grid_spec=grid_spec,
        out_shape=jax.ShapeDtypeStruct((E, 128), jnp.float32),
        compiler_params=pltpu.CompilerParams(
            dimension_semantics=("arbitrary",)),
    )(chunk_start, n_chunks, geo_c, Bm, A_in, rb_in, geoT_in, sbf_W2)


def _update(h, agg, upd_W, upd_b8):
    E = h.shape[0]
    BLK = 512

    def body(h_ref, g_ref, W_ref, b_ref, o_ref):
        o_ref[...] = h_ref[...] + _silu(
            jnp.dot(g_ref[...], W_ref[...],
                    preferred_element_type=jnp.float32) + b_ref[0:1, :])

    return pl.pallas_call(
        body,
        grid=(E // BLK,),
        in_specs=[
            pl.BlockSpec((BLK, 256), lambda i: (i, 0)),
            pl.BlockSpec((BLK, 128), lambda i: (i, 0)),
            pl.BlockSpec((128, 256), lambda i: (0, 0)),
            pl.BlockSpec((8, 256), lambda i: (0, 0)),
        ],
        out_specs=pl.BlockSpec((BLK, 256), lambda i: (i, 0)),
        out_shape=jax.ShapeDtypeStruct((E, 256), jnp.float32),
        compiler_params=pltpu.CompilerParams(
            dimension_semantics=("arbitrary",)),
    )(h, agg, upd_W, upd_b8)


def _edge_out(h, rb, rbf_out_W, out_W, out_b8):
    E = h.shape[0]
    BLK = 512

    def body(h_ref, rb_ref, Wr_ref, Wo_ref, bo_ref, o_ref):
        a = jnp.dot(rb_ref[...], Wr_ref[...],
                    preferred_element_type=jnp.float32)
        b = _silu(jnp.dot(h_ref[...], Wo_ref[...],
                          preferred_element_type=jnp.float32) + bo_ref[0:1, :])
        o_ref[...] = a * b

    return pl.pallas_call(
        body,
        grid=(E // BLK,),
        in_specs=[
            pl.BlockSpec((BLK, 256), lambda i: (i, 0)),
            pl.BlockSpec((BLK, 16), lambda i: (i, 0)),
            pl.BlockSpec((16, 128), lambda i: (0, 0)),
            pl.BlockSpec((256, 128), lambda i: (0, 0)),
            pl.BlockSpec((8, 128), lambda i: (0, 0)),
        ],
        out_specs=pl.BlockSpec((BLK, 128), lambda i: (i, 0)),
        out_shape=jax.ShapeDtypeStruct((E, 128), jnp.float32),
        compiler_params=pltpu.CompilerParams(
            dimension_semantics=("arbitrary",)),
    )(h, rb, rbf_out_W, out_W, out_b8)


def _final(ec0, nec, edge_out, srcfT, a1_W, a1_b8, a2_row8):
    E = edge_out.shape[0]
    NB = 32

    def body(cs_ref, nc_ref, eo_hbm, sT_hbm, W1_ref, b1_ref, a2_ref,
             out_ref, eo_v, sT_v, acc, s1, s2):
        i = pl.program_id(0)

        @pl.when(i == 0)
        def _():
            out_ref[...] = jnp.zeros_like(out_ref)

        c0 = cs_ref[i]
        nc = nc_ref[i]
        acc[...] = jnp.zeros_like(acc)
        rowid = lax.broadcasted_iota(
            jnp.int32, (128, 128), 0).astype(jnp.float32) + (
            jnp.float32(128.0) * i.astype(jnp.float32))

        def chunk(c, carry):
            base = (c0 + c) * 128
            cpE = pltpu.make_async_copy(eo_hbm.at[pl.ds(base, 128), :], eo_v, s1)
            cpS = pltpu.make_async_copy(sT_hbm.at[:, pl.ds(base, 128)], sT_v, s2)
            cpE.start()
            cpS.start()
            cpE.wait()
            cpS.wait()
            srow = sT_v[0:1, :]
            m = jnp.where(rowid == srow, 1.0, 0.0)
            acc[...] = acc[...] + jnp.dot(m, eo_v[...],
                                          preferred_element_type=jnp.float32)
            return carry

        lax.fori_loop(0, nc, chunk, 0)
        z = _silu(jnp.dot(acc[...], W1_ref[...],
                          preferred_element_type=jnp.float32) + b1_ref[0:1, :])
        y = jnp.sum(z * a2_ref[0:1, :], axis=1)
        s = jnp.sum(y)
        oh = jnp.where(
            (lax.broadcasted_iota(jnp.int32, (8, 128), 0) == 0)
            & (lax.broadcasted_iota(jnp.int32, (8, 128), 1) == 0),
            s, 0.0)
        out_ref[...] = out_ref[...] + oh

    grid_spec = pltpu.PrefetchScalarGridSpec(
        num_scalar_prefetch=2,
        grid=(NB,),
        in_specs=[
            pl.BlockSpec(memory_space=pl.ANY),
            pl.BlockSpec(memory_space=pl.ANY),
            pl.BlockSpec((128, 64), lambda i, *_: (0, 0)),
            pl.BlockSpec((8, 64), lambda i, *_: (0, 0)),
            pl.BlockSpec((8, 64), lambda i, *_: (0, 0)),
        ],
        out_specs=pl.BlockSpec((8, 128), lambda i, *_: (0, 0)),
        scratch_shapes=[
            pltpu.VMEM((128, 128), jnp.float32),
            pltpu.VMEM((8, 128), jnp.float32),
            pltpu.VMEM((128, 128), jnp.float32),
            pltpu.SemaphoreType.DMA,
            pltpu.SemaphoreType.DMA,
        ],
    )
    return pl.pallas_call(
        body,
        grid_spec=grid_spec,
        out_shape=jax.ShapeDtypeStruct((8, 128), jnp.float32),
        compiler_params=pltpu.CompilerParams(
            dimension_semantics=("arbitrary",)),
    )(ec0, nec, edge_out, srcfT, a1_W, a1_b8, a2_row8)


def kernel(atom_pos, x, edge_index, edge_attr, emb_table, mat_W, mat_b,
           emb_W, emb_b, sbf_Ws, src_Ws, src_bs, eattr_Ws, eattr_bs,
           upd_Ws, upd_bs, rbf_out_W, out_W, out_b, a1_W, a1_b, a2_W, a2_b):
    N = atom_pos.shape[0]
    E = edge_index.shape[1]
    n_layers = src_Ws.shape[0]
    src = edge_index[0].astype(jnp.int32)
    dst = edge_index[1].astype(jnp.int32)

    p_src = jnp.argsort(src, stable=True)
    srcs = src[p_src]
    dsts = dst[p_src]
    g_dst = jnp.argsort(dst, stable=True)
    q = jnp.zeros((E,), jnp.int32).at[p_src].set(
        jnp.arange(E, dtype=jnp.int32))
    gmap = q[g_dst]

    counts_dst = jnp.bincount(dst, length=N).astype(jnp.int32)
    starts_dst = jnp.concatenate(
        [jnp.zeros((1,), jnp.int32), jnp.cumsum(counts_dst, dtype=jnp.int32)])
    counts_src = jnp.bincount(src, length=N).astype(jnp.int32)
    starts_src = jnp.concatenate(
        [jnp.zeros((1,), jnp.int32), jnp.cumsum(counts_src, dtype=jnp.int32)])

    sm = srcs.reshape(E // 128, 128)
    s_lo = sm[:, 0]
    s_hi = sm[:, -1]
    lo = starts_dst[s_lo]
    hi = starts_dst[s_hi + 1]
    chunk_start = lo // 128
    n_chunks = (hi + 127) // 128 - chunk_start

    ablocks = jnp.arange(N // 128, dtype=jnp.int32)
    e_lo = starts_src[ablocks * 128]
    e_hi = starts_src[(ablocks + 1) * 128]
    ec0 = e_lo // 128
    nec = (e_hi + 127) // 128 - ec0

    ea_pad = jnp.pad(edge_attr, ((0, 0), (0, 14)))
    mat_Wp = jnp.pad(mat_W, ((0, 14), (0, 0)))
    pos_p = jnp.pad(atom_pos, ((0, 0), (0, 13)))
    srcs_f = srcs.astype(jnp.float32)
    dsts_f = dsts.astype(jnp.float32)
    meta_c = jnp.stack(
        [jnp.zeros((E,), jnp.float32)] * 3 + [srcs_f, dsts_f]
        + [jnp.zeros((E,), jnp.float32)] * 11, axis=1)
    x_src_c = x[srcs].astype(jnp.int32)

    def b8(b):
        return jnp.broadcast_to(b[None, :], (8, b.shape[0]))

    sbf_W2 = jnp.transpose(
        sbf_Ws.reshape(n_layers, SBF_DIM, RBF_DIM, 128),
        (0, 2, 1, 3)).reshape(n_layers, RBF_DIM, SBF_DIM * 128)

    ea_c = _sc_gather(ea_pad, p_src)
    pos_s = _sc_gather(pos_p, srcs)
    pos_d = _sc_gather(pos_p, dsts)
    emb_c = _sc_gather(emb_table, x_src_c)

    h, geo_c, rb_c = _stage1(ea_c, pos_s, pos_d, meta_c, mat_Wp,
                             b8(mat_b), emb_W, b8(emb_b))

    geo_rb = jnp.concatenate([geo_c, rb_c], axis=1)
    geo_rb_in = _sc_gather(geo_rb, gmap)
    geoT_in = jnp.transpose(geo_rb_in[:, :16])
    rb_in = geo_rb_in[:, 16:]

    for l in range(n_layers):
        A, Bm = _ab(h, emb_c, src_Ws[l], b8(src_bs[l]),
                    eattr_Ws[l], b8(eattr_bs[l]))
        A_in = _sc_gather(A, gmap)
        agg = _triplet(chunk_start, n_chunks, geo_c, Bm, A_in, rb_in,
                       geoT_in, sbf_W2[l])
        h = _update(h, agg, upd_Ws[l], b8(upd_bs[l]))

    eo = _edge_out(h, rb_c, rbf_out_W, out_W, b8(out_b))

    srcfT = jnp.broadcast_to(srcs_f[None, :], (8, E))
    a2_row8 = jnp.broadcast_to(a2_W.reshape(1, 64), (8, 64))
    res = _final(ec0, nec, eo, srcfT, a1_W, b8(a1_b), a2_row8)
    return res[0:1, 0:1] + jnp.float32(N) * a2_b[0]
